# Initial kernel scaffold; baseline (speedup 1.0000x reference)
import jax
import jax.numpy as jnp
from jax.experimental import pallas as pl


def kernel(x, edge_index, initial_coords, node_w, node_b, coord_w, coord_b, g1_w, g1_b, g2_w, g2_b, g3_w, g3_b, in_proj_w, in_proj_b, out_proj_w, out_proj_b, fh_w, fh_b):
    raise NotImplementedError("write your pallas kernel here")


# speedup vs baseline: 6.3062x; 6.3062x over previous
import functools

import jax
import jax.numpy as jnp
from jax import lax
from jax.experimental import pallas as pl
from jax.experimental.pallas import tpu as pltpu
from jax.experimental.pallas import tpu_sc as plsc

N = 10000
E = 320000
H = 128
NH = 4
HD = 32
NP = 10240

NC = 2
NS = 16
NW = NC * NS
EPW = E // NW
BB = 80
NB = EPW // BB
NA = 10240
RPT = NA // NS


@functools.cache
def _sc_mesh():
    return plsc.VectorSubcoreMesh(core_axis_name="c", subcore_axis_name="s",
                                  num_cores=NC, num_subcores=NS)


@functools.cache
def _sc_edge_agg_kernel():
    return pl.kernel(
        _sc_edge_agg_body,
        out_type=jax.ShapeDtypeStruct((NC, NA, H), jnp.float32),
        mesh=_sc_mesh(),
        scratch_types=[
            pltpu.VMEM((NB, BB), jnp.int32),
            pltpu.VMEM((NB, BB), jnp.int32),
            pltpu.VMEM((BB, H), jnp.float32),
            pltpu.VMEM_SHARED((NA, H), jnp.float32),
            pltpu.SemaphoreType.DMA,
        ],
    )


def _sc_edge_agg_body(m_hbm, src_hbm, dst_hbm, zeros_hbm, out_hbm,
                      src_v, dst_v, rows_v, acc, sem):
    c = lax.axis_index("c")
    s = lax.axis_index("s")
    wid = s * NC + c
    pltpu.sync_copy(zeros_hbm.at[pl.ds(s * RPT, RPT)], acc.at[pl.ds(s * RPT, RPT)])
    pltpu.sync_copy(src_hbm.at[wid], src_v)
    pltpu.sync_copy(dst_hbm.at[wid], dst_v)
    plsc.subcore_barrier()

    def body(j, carry):
        pltpu.async_copy(m_hbm.at[src_v.at[j]], rows_v, sem).wait()
        pltpu.sync_copy(rows_v, acc.at[dst_v.at[j]], add=True)
        return carry

    lax.fori_loop(0, NB, body, 0)
    plsc.subcore_barrier()
    pltpu.sync_copy(acc.at[pl.ds(s * RPT, RPT)], out_hbm.at[c, pl.ds(s * RPT, RPT)])


BR = 1000


def _dinv_block(degp):
    deg = degp[0, :, 0:1] + degp[1, :, 0:1] + 1.0
    return lax.rsqrt(deg)


def _tc_prelude_body(x_ref, cp_ref, degp_ref, nw_ref, nb_ref, cw_ref, cb_ref,
                     g1a_ref, g1c_ref, out_ref):
    h0 = jnp.dot(x_ref[...], nw_ref[...].T, preferred_element_type=jnp.float32) + nb_ref[...]
    hc = jnp.dot(cp_ref[...], cw_ref[...].T, preferred_element_type=jnp.float32) + cb_ref[...]
    dinv = _dinv_block(degp_ref[...])
    m1 = (jnp.dot(h0, g1a_ref[...].T, preferred_element_type=jnp.float32)
          + jnp.dot(hc, g1c_ref[...].T, preferred_element_type=jnp.float32))
    out_ref[...] = dinv * m1


def _tc_prelude(x, cp, degp, nw, nb, cw, cb, g1a, g1c):
    return pl.pallas_call(
        _tc_prelude_body,
        grid=(N // BR,),
        in_specs=[
            pl.BlockSpec((BR, 128), lambda r: (r, 0)),
            pl.BlockSpec((BR, 128), lambda r: (r, 0)),
            pl.BlockSpec((2, BR, H), lambda r: (0, r, 0)),
            pl.BlockSpec((128, 128), lambda r: (0, 0)),
            pl.BlockSpec((1, 128), lambda r: (0, 0)),
            pl.BlockSpec((32, 128), lambda r: (0, 0)),
            pl.BlockSpec((1, 32), lambda r: (0, 0)),
            pl.BlockSpec((128, 128), lambda r: (0, 0)),
            pl.BlockSpec((128, 32), lambda r: (0, 0)),
        ],
        out_specs=pl.BlockSpec((BR, H), lambda r: (r, 0)),
        out_shape=jax.ShapeDtypeStruct((N, H), jnp.float32),
    )(x, cp, degp, nw, nb, cw, cb, g1a, g1c)


def _tc_layer_body(p_ref, m_ref, degp_ref, bprev_ref, w_ref, bout_ref, out_ref,
                   *, scale_out):
    dinv = _dinv_block(degp_ref[...])
    p = p_ref[...]
    h = jnp.maximum(dinv * (p[0] + p[1] + m_ref[...]) + bprev_ref[...], 0.0)
    out = jnp.dot(h, w_ref[...].T, preferred_element_type=jnp.float32)
    if scale_out:
        out = dinv * out
    out_ref[...] = out + bout_ref[...]


def _tc_layer(p, m, degp, bprev, w, bout, scale_out):
    odim = w.shape[0]
    return pl.pallas_call(
        functools.partial(_tc_layer_body, scale_out=scale_out),
        grid=(N // BR,),
        in_specs=[
            pl.BlockSpec((2, BR, H), lambda r: (0, r, 0)),
            pl.BlockSpec((BR, H), lambda r: (r, 0)),
            pl.BlockSpec((2, BR, H), lambda r: (0, r, 0)),
            pl.BlockSpec((1, H), lambda r: (0, 0)),
            pl.BlockSpec((odim, H), lambda r: (0, 0)),
            pl.BlockSpec((1, odim), lambda r: (0, 0)),
        ],
        out_specs=pl.BlockSpec((BR, odim), lambda r: (r, 0)),
        out_shape=jax.ShapeDtypeStruct((N, odim), jnp.float32),
    )(p, m, degp, bprev, w, bout)


BQ = 512
BK = 512
NQB = NP // BQ
NKB = NP // BK
_SCALE = float(1.0 / (float(HD) ** 0.5))


def _tc_flash_body(q_ref, k_ref, v_ref, o_ref):
    q = q_ref[0] * _SCALE

    def body(kb, carry):
        o, m, l = carry
        k = k_ref[0, pl.ds(kb * BK, BK), :]
        v = v_ref[0, pl.ds(kb * BK, BK), :]
        s = lax.dot_general(q, k, (((1,), (1,)), ((), ())),
                            preferred_element_type=jnp.float32)
        col = kb * BK + lax.broadcasted_iota(jnp.int32, (BQ, BK), 1)
        s = jnp.where(col < N, s, -1e30)
        mb = jnp.max(s, axis=1, keepdims=True)
        mnew = jnp.maximum(m, mb)
        pmat = jnp.exp(s - mnew)
        alpha = jnp.exp(m - mnew)
        l = l * alpha + jnp.sum(pmat, axis=1, keepdims=True)
        o = o * alpha + jnp.dot(pmat, v, preferred_element_type=jnp.float32)
        return o, mnew, l

    o0 = jnp.zeros((BQ, HD), jnp.float32)
    m0 = jnp.full((BQ, 1), -1e30, jnp.float32)
    l0 = jnp.zeros((BQ, 1), jnp.float32)
    o, m, l = lax.fori_loop(0, NKB, body, (o0, m0, l0))
    o_ref[0] = o / l


def _tc_flash(q3, k3, v3):
    return pl.pallas_call(
        _tc_flash_body,
        grid=(NH, NQB),
        in_specs=[
            pl.BlockSpec((1, BQ, HD), lambda h, qi: (h, qi, 0)),
            pl.BlockSpec((1, NP, HD), lambda h, qi: (h, 0, 0)),
            pl.BlockSpec((1, NP, HD), lambda h, qi: (h, 0, 0)),
        ],
        out_specs=pl.BlockSpec((1, BQ, HD), lambda h, qi: (h, qi, 0)),
        out_shape=jax.ShapeDtypeStruct((NH, NP, HD), jnp.float32),
    )(q3, k3, v3)


def _tc_epilogue_body(o_ref, w_ref, b_ref, out_ref):
    out_ref[...] = jnp.dot(o_ref[...], w_ref[...],
                           preferred_element_type=jnp.float32) + b_ref[...]


def _tc_epilogue(o, wct, bc):
    return pl.pallas_call(
        _tc_epilogue_body,
        grid=(NP // 1024,),
        in_specs=[
            pl.BlockSpec((1024, H), lambda r: (r, 0)),
            pl.BlockSpec((H, H), lambda r: (0, 0)),
            pl.BlockSpec((1, H), lambda r: (0, 0)),
        ],
        out_specs=pl.BlockSpec((1024, H), lambda r: (r, 0)),
        out_shape=jax.ShapeDtypeStruct((NP, H), jnp.float32),
    )(o, wct, bc)


def kernel(x, edge_index, initial_coords, node_w, node_b, coord_w, coord_b,
           g1_w, g1_b, g2_w, g2_b, g3_w, g3_b,
           in_proj_w, in_proj_b, out_proj_w, out_proj_b, fh_w, fh_b):
    src_r = edge_index[0].reshape(NW, NB, BB)
    dst_r = edge_index[1].reshape(NW, NB, BB)
    zeros128 = jnp.zeros((NA, H), jnp.float32)
    ones128 = jnp.ones((N, H), jnp.float32)

    degp = _sc_edge_agg_kernel()(ones128, src_r, dst_r, zeros128)

    cp = jnp.pad(initial_coords, ((0, 0), (0, 126)))
    cw = jnp.pad(coord_w, ((0, 0), (0, 126)))
    zb = jnp.zeros((1, H), jnp.float32)

    m1 = _tc_prelude(x, cp, degp, node_w, node_b.reshape(1, -1), cw,
                     coord_b.reshape(1, -1), g1_w[:, :128], g1_w[:, 128:])
    p1 = _sc_edge_agg_kernel()(m1, src_r, dst_r, zeros128)
    m2 = _tc_layer(p1, m1, degp, g1_b.reshape(1, -1), g2_w, zb, True)
    p2 = _sc_edge_agg_kernel()(m2, src_r, dst_r, zeros128)
    m3 = _tc_layer(p2, m2, degp, g2_b.reshape(1, -1), g3_w, zb, True)
    p3 = _sc_edge_agg_kernel()(m3, src_r, dst_r, zeros128)
    qkv = _tc_layer(p3, m3, degp, g3_b.reshape(1, -1), in_proj_w,
                    in_proj_b.reshape(1, -1), False)

    qkv_p = jnp.pad(qkv, ((0, NP - N), (0, 0)))
    q3 = qkv_p[:, :128].reshape(NP, NH, HD).transpose(1, 0, 2)
    k3 = qkv_p[:, 128:256].reshape(NP, NH, HD).transpose(1, 0, 2)
    v3 = qkv_p[:, 256:].reshape(NP, NH, HD).transpose(1, 0, 2)
    o3 = _tc_flash(q3, k3, v3)
    o = o3.transpose(1, 0, 2).reshape(NP, H)

    wct = jnp.zeros((H, H), jnp.float32).at[:, :2].set((fh_w @ out_proj_w).T)
    bc = jnp.zeros((1, H), jnp.float32).at[0, :2].set(out_proj_b @ fh_w.T + fh_b)
    out = _tc_epilogue(o, wct, bc)
    return out[:N, :2]


# speedup vs baseline: 7.6673x; 1.2158x over previous
import functools

import jax
import jax.numpy as jnp
from jax import lax
from jax.experimental import pallas as pl
from jax.experimental.pallas import tpu as pltpu
from jax.experimental.pallas import tpu_sc as plsc

N = 10000
E = 320000
H = 128
NH = 4
HD = 32
NP = 10240

NC = 2
NS = 16
NW = NC * NS
EPW = E // NW
BB = 125
NB = EPW // BB
PH = 2
NBP = NB // PH
NA = 10240
RPT = NA // NS


@functools.cache
def _sc_mesh():
    return plsc.VectorSubcoreMesh(core_axis_name="c", subcore_axis_name="s",
                                  num_cores=NC, num_subcores=NS)


@functools.cache
def _sc_degree_kernel():
    return pl.kernel(
        _sc_degree_body,
        out_type=jax.ShapeDtypeStruct((NC, NA, H), jnp.float32),
        mesh=_sc_mesh(),
        scratch_types=[
            pltpu.VMEM((NB, BB), jnp.int32),
            pltpu.VMEM((BB, H), jnp.float32),
            pltpu.VMEM_SHARED((NA, H), jnp.float32),
        ],
    )


def _sc_degree_body(dst_hbm, ones_hbm, zeros_hbm, out_hbm, idx_v, ones_v, acc):
    c = lax.axis_index("c")
    s = lax.axis_index("s")
    wid = s * NC + c
    pltpu.sync_copy(zeros_hbm.at[pl.ds(s * RPT, RPT)], acc.at[pl.ds(s * RPT, RPT)])
    pltpu.sync_copy(dst_hbm.at[wid], idx_v)
    pltpu.sync_copy(ones_hbm, ones_v)
    plsc.subcore_barrier()

    def body(j, carry):
        pltpu.sync_copy(ones_v, acc.at[idx_v.at[j]], add=True)
        return carry

    lax.fori_loop(0, NB, body, 0)
    plsc.subcore_barrier()
    pltpu.sync_copy(acc.at[pl.ds(s * RPT, RPT)], out_hbm.at[c, pl.ds(s * RPT, RPT)])


@functools.cache
def _sc_edge_agg_kernel():
    return pl.kernel(
        _sc_edge_agg_body,
        out_type=jax.ShapeDtypeStruct((NC, NA, H), jnp.float32),
        mesh=_sc_mesh(),
        scratch_types=[
            pltpu.VMEM((NBP, BB), jnp.int32),
            pltpu.VMEM((NBP, BB), jnp.int32),
            pltpu.VMEM((BB, H), jnp.float32),
            pltpu.VMEM((BB, H), jnp.float32),
            pltpu.VMEM_SHARED((NA, H), jnp.float32),
            pltpu.SemaphoreType.DMA,
            pltpu.SemaphoreType.DMA,
        ],
    )


def _sc_edge_agg_body(m_hbm, src_hbm, dst_hbm, zeros_hbm, out_hbm,
                      src_v, dst_v, rows_a, rows_b, acc, sem_a, sem_b):
    c = lax.axis_index("c")
    s = lax.axis_index("s")
    wid = s * NC + c
    pltpu.sync_copy(zeros_hbm.at[pl.ds(s * RPT, RPT)], acc.at[pl.ds(s * RPT, RPT)])
    plsc.subcore_barrier()

    def body(t, carry):
        j0 = t * 2
        j1 = j0 + 1
        c0 = pltpu.async_copy(m_hbm.at[src_v.at[j0]], rows_a, sem_a)
        c1 = pltpu.async_copy(m_hbm.at[src_v.at[j1]], rows_b, sem_b)
        c0.wait()
        pltpu.sync_copy(rows_a, acc.at[dst_v.at[j0]], add=True)
        c1.wait()
        pltpu.sync_copy(rows_b, acc.at[dst_v.at[j1]], add=True)
        return carry

    for p in range(PH):
        pltpu.sync_copy(src_hbm.at[wid, pl.ds(p * NBP, NBP)], src_v)
        pltpu.sync_copy(dst_hbm.at[wid, pl.ds(p * NBP, NBP)], dst_v)
        lax.fori_loop(0, NBP // 2, body, 0)
    plsc.subcore_barrier()
    pltpu.sync_copy(acc.at[pl.ds(s * RPT, RPT)], out_hbm.at[c, pl.ds(s * RPT, RPT)])


BR = 1000


def _dinv_block(degp):
    deg = degp[0, :, 0:1] + degp[1, :, 0:1] + 1.0
    return lax.rsqrt(deg)


def _tc_prelude_body(x_ref, cp_ref, degp_ref, nw_ref, nb_ref, cw_ref, cb_ref,
                     g1a_ref, g1c_ref, out_ref):
    h0 = jnp.dot(x_ref[...], nw_ref[...].T, preferred_element_type=jnp.float32) + nb_ref[...]
    hc = jnp.dot(cp_ref[...], cw_ref[...].T, preferred_element_type=jnp.float32) + cb_ref[...]
    dinv = _dinv_block(degp_ref[...])
    m1 = (jnp.dot(h0, g1a_ref[...].T, preferred_element_type=jnp.float32)
          + jnp.dot(hc, g1c_ref[...].T, preferred_element_type=jnp.float32))
    out_ref[...] = dinv * m1


def _tc_prelude(x, cp, degp, nw, nb, cw, cb, g1a, g1c):
    return pl.pallas_call(
        _tc_prelude_body,
        grid=(N // BR,),
        in_specs=[
            pl.BlockSpec((BR, 128), lambda r: (r, 0)),
            pl.BlockSpec((BR, 128), lambda r: (r, 0)),
            pl.BlockSpec((2, BR, H), lambda r: (0, r, 0)),
            pl.BlockSpec((128, 128), lambda r: (0, 0)),
            pl.BlockSpec((1, 128), lambda r: (0, 0)),
            pl.BlockSpec((32, 128), lambda r: (0, 0)),
            pl.BlockSpec((1, 32), lambda r: (0, 0)),
            pl.BlockSpec((128, 128), lambda r: (0, 0)),
            pl.BlockSpec((128, 32), lambda r: (0, 0)),
        ],
        out_specs=pl.BlockSpec((BR, H), lambda r: (r, 0)),
        out_shape=jax.ShapeDtypeStruct((N, H), jnp.float32),
    )(x, cp, degp, nw, nb, cw, cb, g1a, g1c)


def _tc_layer_body(p_ref, m_ref, degp_ref, bprev_ref, w_ref, bout_ref, out_ref,
                   *, scale_out):
    dinv = _dinv_block(degp_ref[...])
    p = p_ref[...]
    h = jnp.maximum(dinv * (p[0] + p[1] + m_ref[...]) + bprev_ref[...], 0.0)
    out = jnp.dot(h, w_ref[...].T, preferred_element_type=jnp.float32)
    if scale_out:
        out = dinv * out
    out_ref[...] = out + bout_ref[...]


def _tc_layer(p, m, degp, bprev, w, bout, scale_out):
    odim = w.shape[0]
    return pl.pallas_call(
        functools.partial(_tc_layer_body, scale_out=scale_out),
        grid=(N // BR,),
        in_specs=[
            pl.BlockSpec((2, BR, H), lambda r: (0, r, 0)),
            pl.BlockSpec((BR, H), lambda r: (r, 0)),
            pl.BlockSpec((2, BR, H), lambda r: (0, r, 0)),
            pl.BlockSpec((1, H), lambda r: (0, 0)),
            pl.BlockSpec((odim, H), lambda r: (0, 0)),
            pl.BlockSpec((1, odim), lambda r: (0, 0)),
        ],
        out_specs=pl.BlockSpec((BR, odim), lambda r: (r, 0)),
        out_shape=jax.ShapeDtypeStruct((N, odim), jnp.float32),
    )(p, m, degp, bprev, w, bout)


BQ = 512
BK = 512
NQB = NP // BQ
NKB = NP // BK
_SCALE = float(1.0 / (float(HD) ** 0.5))


def _tc_flash_body(q_ref, k_ref, v_ref, o_ref):
    q = (q_ref[0] * _SCALE).astype(jnp.bfloat16)

    def body(kb, carry):
        o, m = carry
        k = k_ref[0, pl.ds(kb * BK, BK), :]
        v = v_ref[0, pl.ds(kb * BK, BK), :]
        s = lax.dot_general(q, k, (((1,), (1,)), ((), ())),
                            preferred_element_type=jnp.float32)
        mb = jnp.max(s, axis=1, keepdims=True)
        mnew = jnp.maximum(m, mb)
        pmat = jnp.exp(s - mnew).astype(jnp.bfloat16)
        alpha = jnp.exp(m - mnew)
        o = o * alpha + jnp.dot(pmat, v, preferred_element_type=jnp.float32)
        return o, mnew

    o0 = jnp.zeros((BQ, 2 * HD), jnp.float32)
    m0 = jnp.full((BQ, 1), -1e30, jnp.float32)
    o, m = lax.fori_loop(0, NKB, body, (o0, m0))
    o_ref[0] = o[:, :HD] / o[:, HD:HD + 1]


def _tc_flash(q3, k3, v3e):
    return pl.pallas_call(
        _tc_flash_body,
        grid=(NH, NQB),
        in_specs=[
            pl.BlockSpec((1, BQ, HD), lambda h, qi: (h, qi, 0)),
            pl.BlockSpec((1, NP, HD), lambda h, qi: (h, 0, 0)),
            pl.BlockSpec((1, NP, 2 * HD), lambda h, qi: (h, 0, 0)),
        ],
        out_specs=pl.BlockSpec((1, BQ, HD), lambda h, qi: (h, qi, 0)),
        out_shape=jax.ShapeDtypeStruct((NH, NP, HD), jnp.float32),
    )(q3, k3, v3e)


def _tc_epilogue_body(o_ref, w_ref, b_ref, out_ref):
    out_ref[...] = jnp.dot(o_ref[...], w_ref[...],
                           preferred_element_type=jnp.float32) + b_ref[...]


def _tc_epilogue(o, wct, bc):
    return pl.pallas_call(
        _tc_epilogue_body,
        grid=(NP // 1024,),
        in_specs=[
            pl.BlockSpec((1024, H), lambda r: (r, 0)),
            pl.BlockSpec((H, H), lambda r: (0, 0)),
            pl.BlockSpec((1, H), lambda r: (0, 0)),
        ],
        out_specs=pl.BlockSpec((1024, H), lambda r: (r, 0)),
        out_shape=jax.ShapeDtypeStruct((NP, H), jnp.float32),
    )(o, wct, bc)


def kernel(x, edge_index, initial_coords, node_w, node_b, coord_w, coord_b,
           g1_w, g1_b, g2_w, g2_b, g3_w, g3_b,
           in_proj_w, in_proj_b, out_proj_w, out_proj_b, fh_w, fh_b):
    src_r = edge_index[0].reshape(NW, NB, BB)
    dst_r = edge_index[1].reshape(NW, NB, BB)
    zeros128 = jnp.zeros((NA, H), jnp.float32)
    ones128 = jnp.ones((BB, H), jnp.float32)

    degp = _sc_degree_kernel()(dst_r, ones128, zeros128)

    cp = jnp.pad(initial_coords, ((0, 0), (0, 126)))
    cw = jnp.pad(coord_w, ((0, 0), (0, 126)))
    zb = jnp.zeros((1, H), jnp.float32)

    m1 = _tc_prelude(x, cp, degp, node_w, node_b.reshape(1, -1), cw,
                     coord_b.reshape(1, -1), g1_w[:, :128], g1_w[:, 128:])
    p1 = _sc_edge_agg_kernel()(m1, src_r, dst_r, zeros128)
    m2 = _tc_layer(p1, m1, degp, g1_b.reshape(1, -1), g2_w, zb, True)
    p2 = _sc_edge_agg_kernel()(m2, src_r, dst_r, zeros128)
    m3 = _tc_layer(p2, m2, degp, g2_b.reshape(1, -1), g3_w, zb, True)
    p3 = _sc_edge_agg_kernel()(m3, src_r, dst_r, zeros128)
    qkv = _tc_layer(p3, m3, degp, g3_b.reshape(1, -1), in_proj_w,
                    in_proj_b.reshape(1, -1), False)

    qkv_p = jnp.pad(qkv, ((0, NP - N), (0, 0)))
    q3 = qkv_p[:, :128].reshape(NP, NH, HD).transpose(1, 0, 2)
    k3 = qkv_p[:, 128:256].reshape(NP, NH, HD).transpose(1, 0, 2).astype(jnp.bfloat16)
    v3 = qkv_p[:, 256:].reshape(NP, NH, HD).transpose(1, 0, 2)
    lcol = (jnp.arange(NP) < N).astype(jnp.float32)
    v3e = jnp.concatenate(
        [v3, jnp.broadcast_to(lcol[None, :, None], (NH, NP, 1)),
         jnp.zeros((NH, NP, HD - 1), jnp.float32)], axis=-1).astype(jnp.bfloat16)
    o3 = _tc_flash(q3, k3, v3e)
    o = o3.transpose(1, 0, 2).reshape(NP, H)

    wct = jnp.zeros((H, H), jnp.float32).at[:, :2].set((fh_w @ out_proj_w).T)
    bc = jnp.zeros((1, H), jnp.float32).at[0, :2].set(out_proj_b @ fh_w.T + fh_b)
    out = _tc_epilogue(o, wct, bc)
    return out[:N, :2]


# speedup vs baseline: 10.0752x; 1.3140x over previous
import functools

import jax
import jax.numpy as jnp
from jax import lax
from jax.experimental import pallas as pl
from jax.experimental.pallas import tpu as pltpu
from jax.experimental.pallas import tpu_sc as plsc

N = 10000
E = 320000
H = 128
NH = 4
HD = 32
NP = 10240

NC = 2
NS = 16
NW = NC * NS
EPW = E // NW
BB = 125
NB = EPW // BB
PH = 2
NBP = NB // PH
NA = 10240
RPT = NA // NS


@functools.cache
def _sc_mesh():
    return plsc.VectorSubcoreMesh(core_axis_name="c", subcore_axis_name="s",
                                  num_cores=NC, num_subcores=NS)


@functools.cache
def _sc_degree_kernel():
    return pl.kernel(
        _sc_degree_body,
        out_type=jax.ShapeDtypeStruct((NC, NA, H), jnp.float32),
        mesh=_sc_mesh(),
        scratch_types=[
            pltpu.VMEM((NB, BB), jnp.int32),
            pltpu.VMEM((BB, H), jnp.float32),
            pltpu.VMEM_SHARED((NA, H), jnp.float32),
        ],
    )


def _sc_degree_body(dst_hbm, ones_hbm, zeros_hbm, out_hbm, idx_v, ones_v, acc):
    c = lax.axis_index("c")
    s = lax.axis_index("s")
    wid = s * NC + c
    pltpu.sync_copy(zeros_hbm.at[pl.ds(s * RPT, RPT)], acc.at[pl.ds(s * RPT, RPT)])
    pltpu.sync_copy(dst_hbm.at[wid], idx_v)
    pltpu.sync_copy(ones_hbm, ones_v)
    plsc.subcore_barrier()

    def body(j, carry):
        pltpu.sync_copy(ones_v, acc.at[idx_v.at[j]], add=True)
        return carry

    lax.fori_loop(0, NB, body, 0)
    plsc.subcore_barrier()
    pltpu.sync_copy(acc.at[pl.ds(s * RPT, RPT)], out_hbm.at[c, pl.ds(s * RPT, RPT)])


@functools.cache
def _sc_edge_agg_kernel():
    return pl.kernel(
        _sc_edge_agg_body,
        out_type=jax.ShapeDtypeStruct((NC, NA, H), jnp.float32),
        mesh=_sc_mesh(),
        scratch_types=[
            pltpu.VMEM((NBP, BB), jnp.int32),
            pltpu.VMEM((NBP, BB), jnp.int32),
            pltpu.VMEM((BB, H), jnp.float32),
            pltpu.VMEM((BB, H), jnp.float32),
            pltpu.VMEM_SHARED((NA, H), jnp.float32),
            pltpu.SemaphoreType.DMA,
            pltpu.SemaphoreType.DMA,
            pltpu.SemaphoreType.DMA,
            pltpu.SemaphoreType.DMA,
        ],
    )


def _sc_edge_agg_body(m_hbm, src_hbm, dst_hbm, zeros_hbm, out_hbm,
                      src_v, dst_v, rows_a, rows_b, acc,
                      gsa, gsb, ssa, ssb):
    c = lax.axis_index("c")
    s = lax.axis_index("s")
    wid = s * NC + c
    pltpu.sync_copy(zeros_hbm.at[pl.ds(s * RPT, RPT)], acc.at[pl.ds(s * RPT, RPT)])
    plsc.subcore_barrier()

    def body(t, carry):
        j0 = t * 2
        j1 = j0 + 1

        @pl.when(t > 0)
        def _drain_prev():
            pltpu.make_async_copy(rows_a, acc.at[dst_v.at[j0]], ssa).wait()
            pltpu.make_async_copy(rows_b, acc.at[dst_v.at[j1]], ssb).wait()

        g0 = pltpu.async_copy(m_hbm.at[src_v.at[j0]], rows_a, gsa)
        g1 = pltpu.async_copy(m_hbm.at[src_v.at[j1]], rows_b, gsb)
        g0.wait()
        pltpu.async_copy(rows_a, acc.at[dst_v.at[j0]], ssa, add=True)
        g1.wait()
        pltpu.async_copy(rows_b, acc.at[dst_v.at[j1]], ssb, add=True)
        return carry

    for p in range(PH):
        pltpu.sync_copy(src_hbm.at[wid, pl.ds(p * NBP, NBP)], src_v)
        pltpu.sync_copy(dst_hbm.at[wid, pl.ds(p * NBP, NBP)], dst_v)
        lax.fori_loop(0, NBP // 2, body, 0)
        pltpu.make_async_copy(rows_a, acc.at[dst_v.at[0]], ssa).wait()
        pltpu.make_async_copy(rows_b, acc.at[dst_v.at[1]], ssb).wait()
    plsc.subcore_barrier()
    pltpu.sync_copy(acc.at[pl.ds(s * RPT, RPT)], out_hbm.at[c, pl.ds(s * RPT, RPT)])


BR = 1000


def _dinv_block(degp):
    deg = degp[0, :, 0:1] + degp[1, :, 0:1] + 1.0
    return lax.rsqrt(deg)


def _tc_prelude_body(x_ref, cp_ref, degp_ref, nw_ref, nb_ref, cw_ref, cb_ref,
                     g1a_ref, g1c_ref, out_ref):
    h0 = jnp.dot(x_ref[...], nw_ref[...].T, preferred_element_type=jnp.float32) + nb_ref[...]
    hc = jnp.dot(cp_ref[...], cw_ref[...].T, preferred_element_type=jnp.float32) + cb_ref[...]
    dinv = _dinv_block(degp_ref[...])
    m1 = (jnp.dot(h0, g1a_ref[...].T, preferred_element_type=jnp.float32)
          + jnp.dot(hc, g1c_ref[...].T, preferred_element_type=jnp.float32))
    out_ref[...] = dinv * m1


def _tc_prelude(x, cp, degp, nw, nb, cw, cb, g1a, g1c):
    return pl.pallas_call(
        _tc_prelude_body,
        grid=(N // BR,),
        in_specs=[
            pl.BlockSpec((BR, 128), lambda r: (r, 0)),
            pl.BlockSpec((BR, 128), lambda r: (r, 0)),
            pl.BlockSpec((2, BR, H), lambda r: (0, r, 0)),
            pl.BlockSpec((128, 128), lambda r: (0, 0)),
            pl.BlockSpec((1, 128), lambda r: (0, 0)),
            pl.BlockSpec((32, 128), lambda r: (0, 0)),
            pl.BlockSpec((1, 32), lambda r: (0, 0)),
            pl.BlockSpec((128, 128), lambda r: (0, 0)),
            pl.BlockSpec((128, 32), lambda r: (0, 0)),
        ],
        out_specs=pl.BlockSpec((BR, H), lambda r: (r, 0)),
        out_shape=jax.ShapeDtypeStruct((N, H), jnp.float32),
    )(x, cp, degp, nw, nb, cw, cb, g1a, g1c)


def _tc_layer_body(p_ref, m_ref, degp_ref, bprev_ref, w_ref, bout_ref, out_ref,
                   *, scale_out):
    dinv = _dinv_block(degp_ref[...])
    p = p_ref[...]
    h = jnp.maximum(dinv * (p[0] + p[1] + m_ref[...]) + bprev_ref[...], 0.0)
    out = jnp.dot(h, w_ref[...].T, preferred_element_type=jnp.float32)
    if scale_out:
        out = dinv * out
    out_ref[...] = out + bout_ref[...]


def _tc_layer(p, m, degp, bprev, w, bout, scale_out):
    odim = w.shape[0]
    return pl.pallas_call(
        functools.partial(_tc_layer_body, scale_out=scale_out),
        grid=(N // BR,),
        in_specs=[
            pl.BlockSpec((2, BR, H), lambda r: (0, r, 0)),
            pl.BlockSpec((BR, H), lambda r: (r, 0)),
            pl.BlockSpec((2, BR, H), lambda r: (0, r, 0)),
            pl.BlockSpec((1, H), lambda r: (0, 0)),
            pl.BlockSpec((odim, H), lambda r: (0, 0)),
            pl.BlockSpec((1, odim), lambda r: (0, 0)),
        ],
        out_specs=pl.BlockSpec((BR, odim), lambda r: (r, 0)),
        out_shape=jax.ShapeDtypeStruct((N, odim), jnp.float32),
    )(p, m, degp, bprev, w, bout)


BQ = 1024
BK = 1024
NQB = NP // BQ
NKB = NP // BK
_SCALE = float(1.0 / (float(HD) ** 0.5))


def _tc_flash_body(q_ref, k_ref, v_ref, o_ref):
    q = (q_ref[0] * _SCALE).astype(jnp.bfloat16)

    def body(kb, carry):
        o, m = carry
        k = k_ref[0, pl.ds(kb * BK, BK), :]
        v = v_ref[0, pl.ds(kb * BK, BK), :]
        s = lax.dot_general(q, k, (((1,), (1,)), ((), ())),
                            preferred_element_type=jnp.float32)
        mb = jnp.max(s, axis=1, keepdims=True)
        mnew = jnp.maximum(m, mb)
        pmat = jnp.exp(s - mnew).astype(jnp.bfloat16)
        alpha = jnp.exp(m - mnew)
        o = o * alpha + jnp.dot(pmat, v, preferred_element_type=jnp.float32)
        return o, mnew

    o0 = jnp.zeros((BQ, 2 * HD), jnp.float32)
    m0 = jnp.full((BQ, 1), -1e30, jnp.float32)
    o, m = lax.fori_loop(0, NKB, body, (o0, m0))
    o_ref[0] = o[:, :HD] / o[:, HD:HD + 1]


def _tc_flash(q3, k3, v3e):
    return pl.pallas_call(
        _tc_flash_body,
        grid=(NH, NQB),
        in_specs=[
            pl.BlockSpec((1, BQ, HD), lambda h, qi: (h, qi, 0)),
            pl.BlockSpec((1, NP, HD), lambda h, qi: (h, 0, 0)),
            pl.BlockSpec((1, NP, 2 * HD), lambda h, qi: (h, 0, 0)),
        ],
        out_specs=pl.BlockSpec((1, BQ, HD), lambda h, qi: (h, qi, 0)),
        out_shape=jax.ShapeDtypeStruct((NH, NP, HD), jnp.float32),
    )(q3, k3, v3e)


def _tc_epilogue_body(o_ref, w_ref, b_ref, out_ref):
    out_ref[...] = jnp.dot(o_ref[...], w_ref[...],
                           preferred_element_type=jnp.float32) + b_ref[...]


def _tc_epilogue(o, wct, bc):
    return pl.pallas_call(
        _tc_epilogue_body,
        grid=(NP // 1024,),
        in_specs=[
            pl.BlockSpec((1024, H), lambda r: (r, 0)),
            pl.BlockSpec((H, H), lambda r: (0, 0)),
            pl.BlockSpec((1, H), lambda r: (0, 0)),
        ],
        out_specs=pl.BlockSpec((1024, H), lambda r: (r, 0)),
        out_shape=jax.ShapeDtypeStruct((NP, H), jnp.float32),
    )(o, wct, bc)


def kernel(x, edge_index, initial_coords, node_w, node_b, coord_w, coord_b,
           g1_w, g1_b, g2_w, g2_b, g3_w, g3_b,
           in_proj_w, in_proj_b, out_proj_w, out_proj_b, fh_w, fh_b):
    src_r = edge_index[0].reshape(NW, NB, BB)
    dst_r = edge_index[1].reshape(NW, NB, BB)
    zeros128 = jnp.zeros((NA, H), jnp.float32)
    ones128 = jnp.ones((BB, H), jnp.float32)

    degp = _sc_degree_kernel()(dst_r, ones128, zeros128)

    cp = jnp.pad(initial_coords, ((0, 0), (0, 126)))
    cw = jnp.pad(coord_w, ((0, 0), (0, 126)))
    zb = jnp.zeros((1, H), jnp.float32)

    m1 = _tc_prelude(x, cp, degp, node_w, node_b.reshape(1, -1), cw,
                     coord_b.reshape(1, -1), g1_w[:, :128], g1_w[:, 128:])
    p1 = _sc_edge_agg_kernel()(m1, src_r, dst_r, zeros128)
    m2 = _tc_layer(p1, m1, degp, g1_b.reshape(1, -1), g2_w, zb, True)
    p2 = _sc_edge_agg_kernel()(m2, src_r, dst_r, zeros128)
    m3 = _tc_layer(p2, m2, degp, g2_b.reshape(1, -1), g3_w, zb, True)
    p3 = _sc_edge_agg_kernel()(m3, src_r, dst_r, zeros128)
    qkv = _tc_layer(p3, m3, degp, g3_b.reshape(1, -1), in_proj_w,
                    in_proj_b.reshape(1, -1), False)

    qkv_p = jnp.pad(qkv, ((0, NP - N), (0, 0)))
    q3 = qkv_p[:, :128].reshape(NP, NH, HD).transpose(1, 0, 2)
    k3 = qkv_p[:, 128:256].reshape(NP, NH, HD).transpose(1, 0, 2).astype(jnp.bfloat16)
    v3 = qkv_p[:, 256:].reshape(NP, NH, HD).transpose(1, 0, 2)
    lcol = (jnp.arange(NP) < N).astype(jnp.float32)
    v3e = jnp.concatenate(
        [v3, jnp.broadcast_to(lcol[None, :, None], (NH, NP, 1)),
         jnp.zeros((NH, NP, HD - 1), jnp.float32)], axis=-1).astype(jnp.bfloat16)
    o3 = _tc_flash(q3, k3, v3e)
    o = o3.transpose(1, 0, 2).reshape(NP, H)

    wct = jnp.zeros((H, H), jnp.float32).at[:, :2].set((fh_w @ out_proj_w).T)
    bc = jnp.zeros((1, H), jnp.float32).at[0, :2].set(out_proj_b @ fh_w.T + fh_b)
    out = _tc_epilogue(o, wct, bc)
    return out[:N, :2]


# speedup vs baseline: 10.6109x; 1.0532x over previous
import functools

import jax
import jax.numpy as jnp
from jax import lax
from jax.experimental import pallas as pl
from jax.experimental.pallas import tpu as pltpu
from jax.experimental.pallas import tpu_sc as plsc

N = 10000
E = 320000
H = 128
NH = 4
HD = 32
NP = 10240

NC = 2
NS = 16
NW = NC * NS
EPW = E // NW
BB = 125
NB = EPW // BB
PH = 2
NBP = NB // PH
NA = 10240
RPT = NA // NS


@functools.cache
def _sc_mesh():
    return plsc.VectorSubcoreMesh(core_axis_name="c", subcore_axis_name="s",
                                  num_cores=NC, num_subcores=NS)


@functools.cache
def _sc_degree_kernel():
    return pl.kernel(
        _sc_degree_body,
        out_type=jax.ShapeDtypeStruct((NC, NA, H), jnp.float32),
        mesh=_sc_mesh(),
        scratch_types=[
            pltpu.VMEM((NB, BB), jnp.int32),
            pltpu.VMEM((BB, H), jnp.float32),
            pltpu.VMEM_SHARED((NA, H), jnp.float32),
        ],
    )


def _sc_degree_body(dst_hbm, ones_hbm, zeros_hbm, out_hbm, idx_v, ones_v, acc):
    c = lax.axis_index("c")
    s = lax.axis_index("s")
    wid = s * NC + c
    pltpu.sync_copy(zeros_hbm.at[pl.ds(s * RPT, RPT)], acc.at[pl.ds(s * RPT, RPT)])
    pltpu.sync_copy(dst_hbm.at[wid], idx_v)
    pltpu.sync_copy(ones_hbm, ones_v)
    plsc.subcore_barrier()

    def body(j, carry):
        pltpu.sync_copy(ones_v, acc.at[idx_v.at[j]], add=True)
        return carry

    lax.fori_loop(0, NB, body, 0)
    plsc.subcore_barrier()
    pltpu.sync_copy(acc.at[pl.ds(s * RPT, RPT)], out_hbm.at[c, pl.ds(s * RPT, RPT)])


@functools.cache
def _sc_edge_agg_kernel():
    return pl.kernel(
        _sc_edge_agg_body,
        out_type=jax.ShapeDtypeStruct((NC, NA, H), jnp.float32),
        mesh=_sc_mesh(),
        scratch_types=[
            pltpu.VMEM((NBP, BB), jnp.int32),
            pltpu.VMEM((NBP, BB), jnp.int32),
            pltpu.VMEM((BB, H), jnp.float32),
            pltpu.VMEM((BB, H), jnp.float32),
            pltpu.VMEM_SHARED((NA, H), jnp.float32),
            pltpu.SemaphoreType.DMA,
            pltpu.SemaphoreType.DMA,
            pltpu.SemaphoreType.DMA,
            pltpu.SemaphoreType.DMA,
        ],
    )


def _sc_edge_agg_body(m_hbm, src_hbm, dst_hbm, zeros_hbm, out_hbm,
                      src_v, dst_v, rows_a, rows_b, acc,
                      gsa, gsb, ssa, ssb):
    c = lax.axis_index("c")
    s = lax.axis_index("s")
    wid = s * NC + c
    pltpu.sync_copy(zeros_hbm.at[pl.ds(s * RPT, RPT)], acc.at[pl.ds(s * RPT, RPT)])
    plsc.subcore_barrier()

    def body(t, carry):
        j0 = t * 2
        j1 = j0 + 1

        @pl.when(t > 0)
        def _drain_prev():
            pltpu.make_async_copy(rows_a, acc.at[dst_v.at[j0]], ssa).wait()
            pltpu.make_async_copy(rows_b, acc.at[dst_v.at[j1]], ssb).wait()

        g0 = pltpu.async_copy(m_hbm.at[src_v.at[j0]], rows_a, gsa)
        g1 = pltpu.async_copy(m_hbm.at[src_v.at[j1]], rows_b, gsb)
        g0.wait()
        pltpu.async_copy(rows_a, acc.at[dst_v.at[j0]], ssa, add=True)
        g1.wait()
        pltpu.async_copy(rows_b, acc.at[dst_v.at[j1]], ssb, add=True)
        return carry

    for p in range(PH):
        pltpu.sync_copy(src_hbm.at[wid, pl.ds(p * NBP, NBP)], src_v)
        pltpu.sync_copy(dst_hbm.at[wid, pl.ds(p * NBP, NBP)], dst_v)
        lax.fori_loop(0, NBP // 2, body, 0)
        pltpu.make_async_copy(rows_a, acc.at[dst_v.at[0]], ssa).wait()
        pltpu.make_async_copy(rows_b, acc.at[dst_v.at[1]], ssb).wait()
    plsc.subcore_barrier()
    pltpu.sync_copy(acc.at[pl.ds(s * RPT, RPT)], out_hbm.at[c, pl.ds(s * RPT, RPT)])


BR = 1000


def _dinv_block(degp):
    deg = degp[0, :, 0:1] + degp[1, :, 0:1] + 1.0
    return lax.rsqrt(deg)


def _tc_prelude_body(x_ref, cp_ref, degp_ref, nw_ref, nb_ref, cw_ref, cb_ref,
                     g1a_ref, g1c_ref, out_ref):
    h0 = jnp.dot(x_ref[...], nw_ref[...].T, preferred_element_type=jnp.float32) + nb_ref[...]
    hc = jnp.dot(cp_ref[...], cw_ref[...].T, preferred_element_type=jnp.float32) + cb_ref[...]
    dinv = _dinv_block(degp_ref[...])
    m1 = (jnp.dot(h0, g1a_ref[...].T, preferred_element_type=jnp.float32)
          + jnp.dot(hc, g1c_ref[...].T, preferred_element_type=jnp.float32))
    out_ref[...] = dinv * m1


def _tc_prelude(x, cp, degp, nw, nb, cw, cb, g1a, g1c):
    return pl.pallas_call(
        _tc_prelude_body,
        grid=(N // BR,),
        in_specs=[
            pl.BlockSpec((BR, 128), lambda r: (r, 0)),
            pl.BlockSpec((BR, 128), lambda r: (r, 0)),
            pl.BlockSpec((2, BR, H), lambda r: (0, r, 0)),
            pl.BlockSpec((128, 128), lambda r: (0, 0)),
            pl.BlockSpec((1, 128), lambda r: (0, 0)),
            pl.BlockSpec((32, 128), lambda r: (0, 0)),
            pl.BlockSpec((1, 32), lambda r: (0, 0)),
            pl.BlockSpec((128, 128), lambda r: (0, 0)),
            pl.BlockSpec((128, 32), lambda r: (0, 0)),
        ],
        out_specs=pl.BlockSpec((BR, H), lambda r: (r, 0)),
        out_shape=jax.ShapeDtypeStruct((N, H), jnp.float32),
    )(x, cp, degp, nw, nb, cw, cb, g1a, g1c)


def _tc_layer_body(p_ref, m_ref, degp_ref, bprev_ref, w_ref, bout_ref, out_ref,
                   *, scale_out):
    dinv = _dinv_block(degp_ref[...])
    p = p_ref[...]
    h = jnp.maximum(dinv * (p[0] + p[1] + m_ref[...]) + bprev_ref[...], 0.0)
    out = jnp.dot(h, w_ref[...].T, preferred_element_type=jnp.float32)
    if scale_out:
        out = dinv * out
    out_ref[...] = out + bout_ref[...]


def _tc_layer(p, m, degp, bprev, w, bout, scale_out):
    odim = w.shape[0]
    return pl.pallas_call(
        functools.partial(_tc_layer_body, scale_out=scale_out),
        grid=(N // BR,),
        in_specs=[
            pl.BlockSpec((2, BR, H), lambda r: (0, r, 0)),
            pl.BlockSpec((BR, H), lambda r: (r, 0)),
            pl.BlockSpec((2, BR, H), lambda r: (0, r, 0)),
            pl.BlockSpec((1, H), lambda r: (0, 0)),
            pl.BlockSpec((odim, H), lambda r: (0, 0)),
            pl.BlockSpec((1, odim), lambda r: (0, 0)),
        ],
        out_specs=pl.BlockSpec((BR, odim), lambda r: (r, 0)),
        out_shape=jax.ShapeDtypeStruct((N, odim), jnp.float32),
    )(p, m, degp, bprev, w, bout)


BQ = 1024
BK = 2048
NQB = NP // BQ
NKB = NP // BK
_SCALE = float(1.0 / (float(HD) ** 0.5))


def _tc_flash_body(q_ref, k_ref, v_ref, o_ref):
    q = (q_ref[0] * _SCALE).astype(jnp.bfloat16)

    def body(kb, carry):
        o, m = carry
        k = k_ref[0, pl.ds(kb * BK, BK), :]
        v = v_ref[0, pl.ds(kb * BK, BK), :]
        s = lax.dot_general(q, k, (((1,), (1,)), ((), ())),
                            preferred_element_type=jnp.float32)
        mb = jnp.max(s, axis=1, keepdims=True)
        mnew = jnp.maximum(m, mb)
        pmat = jnp.exp(s - mnew).astype(jnp.bfloat16)
        alpha = jnp.exp(m - mnew)
        o = o * alpha + jnp.dot(pmat, v, preferred_element_type=jnp.float32)
        return o, mnew

    o0 = jnp.zeros((BQ, 2 * HD), jnp.float32)
    m0 = jnp.full((BQ, 1), -1e30, jnp.float32)
    o, m = lax.fori_loop(0, NKB, body, (o0, m0))
    o_ref[0] = o[:, :HD] / o[:, HD:HD + 1]


def _tc_flash(q3, k3, v3e):
    return pl.pallas_call(
        _tc_flash_body,
        grid=(NH, NQB),
        in_specs=[
            pl.BlockSpec((1, BQ, HD), lambda h, qi: (h, qi, 0)),
            pl.BlockSpec((1, NP, HD), lambda h, qi: (h, 0, 0)),
            pl.BlockSpec((1, NP, 2 * HD), lambda h, qi: (h, 0, 0)),
        ],
        out_specs=pl.BlockSpec((1, BQ, HD), lambda h, qi: (h, qi, 0)),
        out_shape=jax.ShapeDtypeStruct((NH, NP, HD), jnp.float32),
    )(q3, k3, v3e)


def _tc_epilogue_body(o_ref, w_ref, b_ref, out_ref):
    out_ref[...] = jnp.dot(o_ref[...], w_ref[...],
                           preferred_element_type=jnp.float32) + b_ref[...]


def _tc_epilogue(o, wct, bc):
    return pl.pallas_call(
        _tc_epilogue_body,
        grid=(NP // 1024,),
        in_specs=[
            pl.BlockSpec((1024, H), lambda r: (r, 0)),
            pl.BlockSpec((H, H), lambda r: (0, 0)),
            pl.BlockSpec((1, H), lambda r: (0, 0)),
        ],
        out_specs=pl.BlockSpec((1024, H), lambda r: (r, 0)),
        out_shape=jax.ShapeDtypeStruct((NP, H), jnp.float32),
    )(o, wct, bc)


def kernel(x, edge_index, initial_coords, node_w, node_b, coord_w, coord_b,
           g1_w, g1_b, g2_w, g2_b, g3_w, g3_b,
           in_proj_w, in_proj_b, out_proj_w, out_proj_b, fh_w, fh_b):
    src_r = edge_index[0].reshape(NW, NB, BB)
    dst_r = edge_index[1].reshape(NW, NB, BB)
    zeros128 = jnp.zeros((NA, H), jnp.float32)
    ones128 = jnp.ones((BB, H), jnp.float32)

    degp = _sc_degree_kernel()(dst_r, ones128, zeros128)

    cp = jnp.pad(initial_coords, ((0, 0), (0, 126)))
    cw = jnp.pad(coord_w, ((0, 0), (0, 126)))
    zb = jnp.zeros((1, H), jnp.float32)

    m1 = _tc_prelude(x, cp, degp, node_w, node_b.reshape(1, -1), cw,
                     coord_b.reshape(1, -1), g1_w[:, :128], g1_w[:, 128:])
    p1 = _sc_edge_agg_kernel()(m1, src_r, dst_r, zeros128)
    m2 = _tc_layer(p1, m1, degp, g1_b.reshape(1, -1), g2_w, zb, True)
    p2 = _sc_edge_agg_kernel()(m2, src_r, dst_r, zeros128)
    m3 = _tc_layer(p2, m2, degp, g2_b.reshape(1, -1), g3_w, zb, True)
    p3 = _sc_edge_agg_kernel()(m3, src_r, dst_r, zeros128)
    qkv = _tc_layer(p3, m3, degp, g3_b.reshape(1, -1), in_proj_w,
                    in_proj_b.reshape(1, -1), False)

    qkv_p = jnp.pad(qkv, ((0, NP - N), (0, 0)))
    q3 = qkv_p[:, :128].reshape(NP, NH, HD).transpose(1, 0, 2)
    k3 = qkv_p[:, 128:256].reshape(NP, NH, HD).transpose(1, 0, 2).astype(jnp.bfloat16)
    v3 = qkv_p[:, 256:].reshape(NP, NH, HD).transpose(1, 0, 2)
    lcol = (jnp.arange(NP) < N).astype(jnp.float32)
    v3e = jnp.concatenate(
        [v3, jnp.broadcast_to(lcol[None, :, None], (NH, NP, 1)),
         jnp.zeros((NH, NP, HD - 1), jnp.float32)], axis=-1).astype(jnp.bfloat16)
    o3 = _tc_flash(q3, k3, v3e)
    o = o3.transpose(1, 0, 2).reshape(NP, H)

    wct = jnp.zeros((H, H), jnp.float32).at[:, :2].set((fh_w @ out_proj_w).T)
    bc = jnp.zeros((1, H), jnp.float32).at[0, :2].set(out_proj_b @ fh_w.T + fh_b)
    out = _tc_epilogue(o, wct, bc)
    return out[:N, :2]
